# Initial kernel scaffold; baseline (speedup 1.0000x reference)
import jax
import jax.numpy as jnp
from jax.experimental import pallas as pl


def kernel(x, edge_index, bn_gamma, bn_beta, W1, b1, W2, b2, W3, b3, Wl, bl):
    raise NotImplementedError("write your pallas kernel here")


# speedup vs baseline: 12.0303x; 12.0303x over previous
import functools

import jax
import jax.numpy as jnp
from jax import lax
from jax.experimental import pallas as pl
from jax.experimental.pallas import tpu as pltpu
from jax.experimental.pallas import tpu_sc as plsc

_N = 10000
_E = 320000
_D = 128
_NC = 2
_NS = 16
_NW = _NC * _NS
_EPW = _E // _NW
_BLK = 128
_NBLK = _E // _BLK
_BASE_BLKS = _NBLK // _NW
_EXTRA_WORKERS = _NBLK % _NW
_NPAD = 10240
_RPS = _NPAD // _NS

_mesh = plsc.VectorSubcoreMesh(
    core_axis_name="c", subcore_axis_name="s", num_cores=_NC, num_subcores=_NS
)


def _deg_body(dst_hbm, out_hbm, idx_v, loc_v):
    c = lax.axis_index("c")
    s = lax.axis_index("s")
    wid = c * _NS + s

    def zero_body(i, carry):
        loc_v[pl.ds(i * 16, 16)] = jnp.zeros((16,), jnp.float32)
        return carry

    lax.fori_loop(0, _N // 16, zero_body, 0)

    pltpu.sync_copy(dst_hbm.at[pl.ds(wid * _EPW, _EPW)], idx_v)
    ones = jnp.ones((16,), jnp.float32)

    def hist_body(i, carry):
        idx = idx_v[pl.ds(i * 16, 16)]
        plsc.addupdate_scatter(loc_v, [idx], ones)
        return carry

    lax.fori_loop(0, _EPW // 16, hist_body, 0)
    pltpu.sync_copy(loc_v, out_hbm.at[wid])


_deg_call = pl.kernel(
    _deg_body,
    out_type=jax.ShapeDtypeStruct((_NW, _N), jnp.float32),
    mesh=_mesh,
    compiler_params=pltpu.CompilerParams(needs_layout_passes=False),
    scratch_types=[
        pltpu.VMEM((_EPW,), jnp.int32),
        pltpu.VMEM((_N,), jnp.float32),
    ],
)


def _scatter_body(hp_hbm, src_hbm, dst_hbm, zeros_hbm, out_hbm,
                  srcidx_v, dstidx_v, rows_v, acc_sh, sem):
    c = lax.axis_index("c")
    s = lax.axis_index("s")
    wid = c * _NS + s

    pltpu.sync_copy(zeros_hbm.at[pl.ds(s * _RPS, _RPS)],
                    acc_sh.at[pl.ds(s * _RPS, _RPS)])
    plsc.subcore_barrier()

    nblk = _BASE_BLKS + jnp.where(wid < _EXTRA_WORKERS, 1, 0)

    def body(i, carry):
        base = (wid + i * _NW) * _BLK
        pltpu.sync_copy(src_hbm.at[pl.ds(base, _BLK)], srcidx_v)
        pltpu.sync_copy(dst_hbm.at[pl.ds(base, _BLK)], dstidx_v)
        pltpu.async_copy(hp_hbm.at[srcidx_v], rows_v, sem).wait()
        pltpu.sync_copy(rows_v, acc_sh.at[dstidx_v], add=True)
        return carry

    lax.fori_loop(0, nblk, body, 0)
    plsc.subcore_barrier()
    pltpu.sync_copy(acc_sh.at[pl.ds(s * _RPS, _RPS)],
                    out_hbm.at[c, pl.ds(s * _RPS, _RPS)])


_scatter_call = pl.kernel(
    _scatter_body,
    out_type=jax.ShapeDtypeStruct((_NC, _NPAD, _D), jnp.float32),
    mesh=_mesh,
    compiler_params=pltpu.CompilerParams(needs_layout_passes=False),
    scratch_types=[
        pltpu.VMEM((_BLK,), jnp.int32),
        pltpu.VMEM((_BLK,), jnp.int32),
        pltpu.VMEM((_BLK, _D), jnp.float32),
        pltpu.VMEM_SHARED((_NPAD, _D), jnp.float32),
        pltpu.SemaphoreType.DMA,
    ],
)


_RB = 1000
_GRID = _N // _RB


def _tc_first_body(x_ref, deg_ref, gb_ref, w_ref, hp_ref, dinv_ref):
    deg = jnp.sum(deg_ref[...], axis=0) + 1.0
    dinv = lax.rsqrt(deg)
    h = x_ref[...] * gb_ref[0:1, :] + gb_ref[1:2, :]
    hp = jnp.dot(h, w_ref[...], preferred_element_type=jnp.float32)
    hp_ref[...] = hp * dinv
    dinv_ref[...] = dinv


def _tc_first(x, deg_p, gb, w1):
    return pl.pallas_call(
        _tc_first_body,
        grid=(_GRID,),
        in_specs=[
            pl.BlockSpec((_RB, _D), lambda i: (i, 0)),
            pl.BlockSpec((_NW, _RB, 1), lambda i: (0, i, 0)),
            pl.BlockSpec((2, _D), lambda i: (0, 0)),
            pl.BlockSpec((_D, _D), lambda i: (0, 0)),
        ],
        out_specs=[
            pl.BlockSpec((_RB, _D), lambda i: (i, 0)),
            pl.BlockSpec((_RB, 1), lambda i: (i, 0)),
        ],
        out_shape=[
            jax.ShapeDtypeStruct((_N, _D), jnp.float32),
            jax.ShapeDtypeStruct((_N, 1), jnp.float32),
        ],
    )(x, deg_p, gb, w1)


def _tc_mid_body(acc_ref, hp_ref, dinv_ref, b_ref, w_ref, out_ref):
    dinv = dinv_ref[...]
    z = (acc_ref[0] + acc_ref[1] + hp_ref[...]) * dinv + b_ref[...]
    h = jnp.maximum(z, 0.0)
    out_ref[...] = jnp.dot(h, w_ref[...], preferred_element_type=jnp.float32) * dinv


def _tc_mid(acc_p, hp, dinv, b, w_next):
    return pl.pallas_call(
        _tc_mid_body,
        grid=(_GRID,),
        in_specs=[
            pl.BlockSpec((_NC, _RB, _D), lambda i: (0, i, 0)),
            pl.BlockSpec((_RB, _D), lambda i: (i, 0)),
            pl.BlockSpec((_RB, 1), lambda i: (i, 0)),
            pl.BlockSpec((1, _D), lambda i: (0, 0)),
            pl.BlockSpec((_D, _D), lambda i: (0, 0)),
        ],
        out_specs=pl.BlockSpec((_RB, _D), lambda i: (i, 0)),
        out_shape=jax.ShapeDtypeStruct((_N, _D), jnp.float32),
    )(acc_p, hp, dinv, b, w_next)


def _tc_final_body(acc_ref, hp_ref, dinv_ref, b_ref, wl_ref, bl_ref,
                   out_ref, sum_ref):
    i = pl.program_id(0)
    z = (acc_ref[0] + acc_ref[1] + hp_ref[...]) * dinv_ref[...] + b_ref[...]
    h = jnp.maximum(z, 0.0)
    part = jnp.sum(h, axis=0, keepdims=True)

    @pl.when(i == 0)
    def _():
        sum_ref[...] = part

    @pl.when(i > 0)
    def _():
        sum_ref[...] += part

    @pl.when(i == _GRID - 1)
    def _():
        m = sum_ref[...] * (1.0 / _N)
        logits = jnp.dot(m, wl_ref[...], preferred_element_type=jnp.float32)
        logits = logits + bl_ref[...]
        zmax = jnp.max(logits, axis=1, keepdims=True)
        e = jnp.exp(logits - zmax)
        out_ref[...] = e / jnp.sum(e, axis=1, keepdims=True)


def _tc_final(acc_p, hp, dinv, b, wl, bl):
    return pl.pallas_call(
        _tc_final_body,
        grid=(_GRID,),
        in_specs=[
            pl.BlockSpec((_NC, _RB, _D), lambda i: (0, i, 0)),
            pl.BlockSpec((_RB, _D), lambda i: (i, 0)),
            pl.BlockSpec((_RB, 1), lambda i: (i, 0)),
            pl.BlockSpec((1, _D), lambda i: (0, 0)),
            pl.BlockSpec((_D, 2), lambda i: (0, 0)),
            pl.BlockSpec((1, 2), lambda i: (0, 0)),
        ],
        out_specs=pl.BlockSpec((1, 2), lambda i: (0, 0)),
        out_shape=jax.ShapeDtypeStruct((1, 2), jnp.float32),
        scratch_shapes=[pltpu.VMEM((1, _D), jnp.float32)],
    )(acc_p, hp, dinv, b, wl, bl)


def kernel(x, edge_index, bn_gamma, bn_beta, W1, b1, W2, b2, W3, b3, Wl, bl):
    ei = edge_index.astype(jnp.int32)
    src = ei[0]
    dst = ei[1]

    eps = 1e-5
    gb = jnp.stack([bn_gamma * (1.0 / jnp.sqrt(1.0 + eps)), bn_beta])

    deg_p = _deg_call(dst).reshape(_NW, _N, 1)
    zeros = jnp.zeros((_NPAD, _D), jnp.float32)

    hp1, dinv = _tc_first(x, deg_p, gb, W1)
    acc1 = _scatter_call(hp1, src, dst, zeros)
    hp2 = _tc_mid(acc1, hp1, dinv, b1.reshape(1, _D), W2)
    acc2 = _scatter_call(hp2, src, dst, zeros)
    hp3 = _tc_mid(acc2, hp2, dinv, b2.reshape(1, _D), W3)
    acc3 = _scatter_call(hp3, src, dst, zeros)
    return _tc_final(acc3, hp3, dinv, b3.reshape(1, _D), Wl,
                     bl.reshape(1, 2))
